# baseline (speedup 1.0000x reference)
import functools

import jax
import jax.numpy as jnp
from jax import lax
from jax.experimental import pallas as pl
from jax.experimental.pallas import tpu as pltpu
from jax.experimental.pallas import tpu_sc as plsc


def _gather_body(num_cores, b_per_w, table_hbm, idx_hbm, out_hbm,
                 idx_v, rows_v, sem):
    wid = lax.axis_index("s") * num_cores + lax.axis_index("c")
    base = wid * b_per_w
    pltpu.sync_copy(idx_hbm.at[pl.ds(base, b_per_w)], idx_v)
    pltpu.async_copy(table_hbm.at[idx_v], rows_v, sem).wait()
    pltpu.sync_copy(rows_v, out_hbm.at[pl.ds(base, b_per_w)])


def _sc_gather_packed(table2, idx):
    V2, D2 = table2.shape
    B = idx.shape[0]
    info = plsc.get_sparse_core_info()
    nw = info.num_cores * info.num_subcores
    b_per_w = B // nw
    mesh = plsc.VectorSubcoreMesh(core_axis_name="c", subcore_axis_name="s")
    kern = pl.kernel(
        functools.partial(_gather_body, info.num_cores, b_per_w),
        mesh=mesh,
        out_type=jax.ShapeDtypeStruct((B, D2), jnp.float32),
        scratch_types=[
            pltpu.VMEM((b_per_w,), jnp.int32),
            pltpu.VMEM((b_per_w, D2), jnp.float32),
            pltpu.SemaphoreType.DMA,
        ],
        compiler_params=pltpu.CompilerParams(use_tc_tiling_on_sc=True),
    )
    return kern(table2, idx)


_VT = 2048
_K = 4


def _mm_ring_body(nsteps, tail, wt_ref, e2_ref, b_ref, o_hbm,
                  acc, sems):
    i = pl.program_id(0)
    slot = lax.rem(i, _K)

    @pl.when(i >= _K)
    def _wait_prev():
        pltpu.make_async_copy(
            acc.at[slot], o_hbm.at[pl.ds((i - _K) * _VT, _VT), :],
            sems.at[slot]).wait()

    waug = jnp.concatenate([wt_ref[...], b_ref[...]], axis=0)
    e = e2_ref[:, :64]
    eaug = jnp.concatenate(
        [e, jnp.ones((e.shape[0], 1), jnp.float32)], axis=1)
    acc[slot] = lax.dot_general(
        waug, eaug, (((0,), (1,)), ((), ())),
        preferred_element_type=jnp.float32)

    @pl.when(i < nsteps - 1)
    def _start_full():
        pltpu.make_async_copy(
            acc.at[slot], o_hbm.at[pl.ds(i * _VT, _VT), :],
            sems.at[slot]).start()

    @pl.when(i == nsteps - 1)
    def _tail_and_drain():
        pltpu.make_async_copy(
            acc.at[slot, pl.ds(0, tail), :],
            o_hbm.at[pl.ds(i * _VT, tail), :], sems.at[slot]).start()
        last_slot = (nsteps - 1) % _K
        for k in range(_K):
            if k == last_slot:
                pltpu.make_async_copy(
                    acc.at[k, pl.ds(0, tail), :],
                    o_hbm.at[pl.ds(i * _VT, tail), :], sems.at[k]).wait()
            else:
                pltpu.make_async_copy(
                    acc.at[k], o_hbm.at[pl.ds(0, _VT), :], sems.at[k]).wait()


def _tc_logits_t(embed2, Wt, b2):
    D, V = Wt.shape
    B = embed2.shape[0]
    nsteps = pl.cdiv(V, _VT)
    tail = V - (nsteps - 1) * _VT
    return pl.pallas_call(
        functools.partial(_mm_ring_body, nsteps, tail),
        grid=(nsteps,),
        in_specs=[
            pl.BlockSpec((D, _VT), lambda i: (0, i)),
            pl.BlockSpec((B, 128), lambda i: (0, 0)),
            pl.BlockSpec((1, _VT), lambda i: (0, i)),
        ],
        out_specs=pl.BlockSpec(memory_space=pl.ANY),
        out_shape=jax.ShapeDtypeStruct((V, B), jnp.float32),
        scratch_shapes=[
            pltpu.VMEM((_K, _VT, B), jnp.float32),
            pltpu.SemaphoreType.DMA((_K,)),
        ],
    )(Wt, embed2, b2)


def kernel(x, emb_table, W, b):
    idx = x.reshape(-1).astype(jnp.int32)
    table_pad = jnp.pad(emb_table, ((0, 0), (0, 64)))
    embed2 = _sc_gather_packed(table_pad, idx)
    ot = _tc_logits_t(embed2, W.T, b.reshape(1, -1))
    return ot.T


# speedup vs baseline: 1.0206x; 1.0206x over previous
import functools

import jax
import jax.numpy as jnp
from jax import lax
from jax.experimental import pallas as pl
from jax.experimental.pallas import tpu as pltpu
from jax.experimental.pallas import tpu_sc as plsc


def _gather_body(num_cores, b_per_w, table_hbm, idx_hbm, out_hbm,
                 idx_v, rows_v, sem):
    wid = lax.axis_index("s") * num_cores + lax.axis_index("c")
    base = wid * b_per_w
    pltpu.sync_copy(idx_hbm.at[pl.ds(base, b_per_w)], idx_v)
    pltpu.async_copy(table_hbm.at[idx_v], rows_v, sem).wait()
    pltpu.sync_copy(rows_v, out_hbm.at[pl.ds(base, b_per_w)])


def _sc_gather_packed(table2, idx):
    V2, D2 = table2.shape
    B = idx.shape[0]
    info = plsc.get_sparse_core_info()
    nw = info.num_cores * info.num_subcores
    b_per_w = B // nw
    mesh = plsc.VectorSubcoreMesh(core_axis_name="c", subcore_axis_name="s")
    kern = pl.kernel(
        functools.partial(_gather_body, info.num_cores, b_per_w),
        mesh=mesh,
        out_type=jax.ShapeDtypeStruct((B, D2), jnp.float32),
        scratch_types=[
            pltpu.VMEM((b_per_w,), jnp.int32),
            pltpu.VMEM((b_per_w, D2), jnp.float32),
            pltpu.SemaphoreType.DMA,
        ],
        compiler_params=pltpu.CompilerParams(use_tc_tiling_on_sc=True),
    )
    return kern(table2, idx)


_TPV = 2048


def _tpad_body(et_ref, o_ref):
    d = et_ref.shape[0]
    eye = (lax.broadcasted_iota(jnp.int32, (d, d), 0) ==
           lax.broadcasted_iota(jnp.int32, (d, d), 1)).astype(jnp.float32)
    tr = lax.dot_general(et_ref[...], eye, (((0,), (0,)), ((), ())),
                         preferred_element_type=jnp.float32)
    o_ref[...] = jnp.concatenate(
        [tr, jnp.zeros((tr.shape[0], 128 - d), jnp.float32)], axis=1)


def _tc_transpose_pad(Et):
    D, V = Et.shape
    return pl.pallas_call(
        _tpad_body,
        grid=(pl.cdiv(V, _TPV),),
        in_specs=[pl.BlockSpec((D, _TPV), lambda i: (0, i))],
        out_specs=pl.BlockSpec((_TPV, 128), lambda i: (i, 0)),
        out_shape=jax.ShapeDtypeStruct((V, 128), jnp.float32),
    )(Et)


_VT = 2048
_K = 4


def _mm_ring_body(nsteps, tail, wt_ref, e2_ref, b_ref, o_hbm,
                  acc, sems):
    i = pl.program_id(0)
    slot = lax.rem(i, _K)

    @pl.when(i >= _K)
    def _wait_prev():
        pltpu.make_async_copy(
            acc.at[slot], o_hbm.at[pl.ds((i - _K) * _VT, _VT), :],
            sems.at[slot]).wait()

    waug = jnp.concatenate([wt_ref[...], b_ref[...]], axis=0)
    e = e2_ref[:, :64]
    eaug = jnp.concatenate(
        [e, jnp.ones((e.shape[0], 1), jnp.float32)], axis=1)
    acc[slot] = lax.dot_general(
        waug, eaug, (((0,), (1,)), ((), ())),
        preferred_element_type=jnp.float32)

    @pl.when(i < nsteps - 1)
    def _start_full():
        pltpu.make_async_copy(
            acc.at[slot], o_hbm.at[pl.ds(i * _VT, _VT), :],
            sems.at[slot]).start()

    @pl.when(i == nsteps - 1)
    def _tail_and_drain():
        pltpu.make_async_copy(
            acc.at[slot, pl.ds(0, tail), :],
            o_hbm.at[pl.ds(i * _VT, tail), :], sems.at[slot]).start()
        last_slot = (nsteps - 1) % _K
        for k in range(_K):
            if k == last_slot:
                pltpu.make_async_copy(
                    acc.at[k, pl.ds(0, tail), :],
                    o_hbm.at[pl.ds(i * _VT, tail), :], sems.at[k]).wait()
            else:
                pltpu.make_async_copy(
                    acc.at[k], o_hbm.at[pl.ds(0, _VT), :], sems.at[k]).wait()


def _tc_logits_t(embed2, Wt, b2):
    D, V = Wt.shape
    B = embed2.shape[0]
    nsteps = pl.cdiv(V, _VT)
    tail = V - (nsteps - 1) * _VT
    return pl.pallas_call(
        functools.partial(_mm_ring_body, nsteps, tail),
        grid=(nsteps,),
        in_specs=[
            pl.BlockSpec((D, _VT), lambda i: (0, i)),
            pl.BlockSpec((B, 128), lambda i: (0, 0)),
            pl.BlockSpec((1, _VT), lambda i: (0, i)),
        ],
        out_specs=pl.BlockSpec(memory_space=pl.ANY),
        out_shape=jax.ShapeDtypeStruct((V, B), jnp.float32),
        scratch_shapes=[
            pltpu.VMEM((_K, _VT, B), jnp.float32),
            pltpu.SemaphoreType.DMA((_K,)),
        ],
    )(Wt, embed2, b2)


def kernel(x, emb_table, W, b):
    idx = x.reshape(-1).astype(jnp.int32)
    table_pad = _tc_transpose_pad(emb_table.T)
    embed2 = _sc_gather_packed(table_pad, idx)
    ot = _tc_logits_t(embed2, W.T, b.reshape(1, -1))
    return ot.T


# speedup vs baseline: 1.1584x; 1.1351x over previous
import functools

import jax
import jax.numpy as jnp
from jax import lax
from jax.experimental import pallas as pl
from jax.experimental.pallas import tpu as pltpu
from jax.experimental.pallas import tpu_sc as plsc


def _gather_body(num_cores, b_per_w, table_hbm, idx_hbm, out_hbm,
                 idx_v, rows_v, sem):
    wid = lax.axis_index("s") * num_cores + lax.axis_index("c")
    base = wid * b_per_w
    pltpu.sync_copy(idx_hbm.at[pl.ds(base, b_per_w)], idx_v)
    pltpu.async_copy(table_hbm.at[idx_v], rows_v, sem).wait()
    pltpu.sync_copy(rows_v, out_hbm.at[pl.ds(base, b_per_w)])


def _sc_gather_packed(table2, idx):
    V2, D2 = table2.shape
    B = idx.shape[0]
    info = plsc.get_sparse_core_info()
    nw = info.num_cores * info.num_subcores
    b_per_w = B // nw
    mesh = plsc.VectorSubcoreMesh(core_axis_name="c", subcore_axis_name="s")
    kern = pl.kernel(
        functools.partial(_gather_body, info.num_cores, b_per_w),
        mesh=mesh,
        out_type=jax.ShapeDtypeStruct((B, D2), jnp.float32),
        scratch_types=[
            pltpu.VMEM((b_per_w,), jnp.int32),
            pltpu.VMEM((b_per_w, D2), jnp.float32),
            pltpu.SemaphoreType.DMA,
        ],
        compiler_params=pltpu.CompilerParams(use_tc_tiling_on_sc=True),
    )
    return kern(table2, idx)


_TPV = 8192
_TK = 3


def _tpad_body(nsteps, tail, et_ref, o_hbm, acc, sems):
    i = pl.program_id(0)
    slot = lax.rem(i, _TK)

    @pl.when(i >= _TK)
    def _wait_prev():
        pltpu.make_async_copy(
            acc.at[slot], o_hbm.at[pl.ds((i - _TK) * _TPV, _TPV), :],
            sems.at[slot]).wait()

    d = et_ref.shape[0]
    tr = lax.transpose(et_ref[...], (1, 0))
    acc[slot] = jnp.concatenate(
        [tr, jnp.zeros((tr.shape[0], 128 - d), jnp.float32)], axis=1)

    @pl.when(i < nsteps - 1)
    def _start_full():
        pltpu.make_async_copy(
            acc.at[slot], o_hbm.at[pl.ds(i * _TPV, _TPV), :],
            sems.at[slot]).start()

    @pl.when(i == nsteps - 1)
    def _tail_and_drain():
        pltpu.make_async_copy(
            acc.at[slot, pl.ds(0, tail), :],
            o_hbm.at[pl.ds(i * _TPV, tail), :], sems.at[slot]).start()
        last_slot = (nsteps - 1) % _TK
        for k in range(_TK):
            if k == last_slot:
                pltpu.make_async_copy(
                    acc.at[k, pl.ds(0, tail), :],
                    o_hbm.at[pl.ds(i * _TPV, tail), :], sems.at[k]).wait()
            else:
                pltpu.make_async_copy(
                    acc.at[k], o_hbm.at[pl.ds(0, _TPV), :], sems.at[k]).wait()


def _tc_transpose_pad(Et):
    D, V = Et.shape
    nsteps = pl.cdiv(V, _TPV)
    tail = V - (nsteps - 1) * _TPV
    return pl.pallas_call(
        functools.partial(_tpad_body, nsteps, tail),
        grid=(nsteps,),
        in_specs=[pl.BlockSpec((D, _TPV), lambda i: (0, i))],
        out_specs=pl.BlockSpec(memory_space=pl.ANY),
        out_shape=jax.ShapeDtypeStruct((V, 128), jnp.float32),
        scratch_shapes=[
            pltpu.VMEM((_TK, _TPV, 128), jnp.float32),
            pltpu.SemaphoreType.DMA((_TK,)),
        ],
    )(Et)


_VT = 2048
_K = 4


def _mm_ring_body(nsteps, tail, wt_ref, e2_ref, b_ref, o_hbm,
                  acc, sems):
    i = pl.program_id(0)
    slot = lax.rem(i, _K)

    @pl.when(i >= _K)
    def _wait_prev():
        pltpu.make_async_copy(
            acc.at[slot], o_hbm.at[pl.ds((i - _K) * _VT, _VT), :],
            sems.at[slot]).wait()

    waug = jnp.concatenate([wt_ref[...], b_ref[...]], axis=0)
    e = e2_ref[:, :64]
    eaug = jnp.concatenate(
        [e, jnp.ones((e.shape[0], 1), jnp.float32)], axis=1)
    acc[slot] = lax.dot_general(
        waug, eaug, (((0,), (1,)), ((), ())),
        preferred_element_type=jnp.float32)

    @pl.when(i < nsteps - 1)
    def _start_full():
        pltpu.make_async_copy(
            acc.at[slot], o_hbm.at[pl.ds(i * _VT, _VT), :],
            sems.at[slot]).start()

    @pl.when(i == nsteps - 1)
    def _tail_and_drain():
        pltpu.make_async_copy(
            acc.at[slot, pl.ds(0, tail), :],
            o_hbm.at[pl.ds(i * _VT, tail), :], sems.at[slot]).start()
        last_slot = (nsteps - 1) % _K
        for k in range(_K):
            if k == last_slot:
                pltpu.make_async_copy(
                    acc.at[k, pl.ds(0, tail), :],
                    o_hbm.at[pl.ds(i * _VT, tail), :], sems.at[k]).wait()
            else:
                pltpu.make_async_copy(
                    acc.at[k], o_hbm.at[pl.ds(0, _VT), :], sems.at[k]).wait()


def _tc_logits_t(embed2, Wt, b2):
    D, V = Wt.shape
    B = embed2.shape[0]
    nsteps = pl.cdiv(V, _VT)
    tail = V - (nsteps - 1) * _VT
    return pl.pallas_call(
        functools.partial(_mm_ring_body, nsteps, tail),
        grid=(nsteps,),
        in_specs=[
            pl.BlockSpec((D, _VT), lambda i: (0, i)),
            pl.BlockSpec((B, 128), lambda i: (0, 0)),
            pl.BlockSpec((1, _VT), lambda i: (0, i)),
        ],
        out_specs=pl.BlockSpec(memory_space=pl.ANY),
        out_shape=jax.ShapeDtypeStruct((V, B), jnp.float32),
        scratch_shapes=[
            pltpu.VMEM((_K, _VT, B), jnp.float32),
            pltpu.SemaphoreType.DMA((_K,)),
        ],
    )(Wt, embed2, b2)


def kernel(x, emb_table, W, b):
    idx = x.reshape(-1).astype(jnp.int32)
    table_pad = _tc_transpose_pad(emb_table.T)
    embed2 = _sc_gather_packed(table_pad, idx)
    ot = _tc_logits_t(embed2, W.T, b.reshape(1, -1))
    return ot.T
